# baseline (speedup 1.0000x reference)
import functools

import jax
import jax.numpy as jnp
from jax import lax
from jax.experimental import pallas as pl
from jax.experimental.pallas import tpu as pltpu
from jax.experimental.pallas import tpu_sc as plsc

B = 16384
F = 128
NC, NS, L = 2, 16, 16
NW = NC * NS
BPW = B // NW
CH = 128
NCHUNK = BPW // CH
GPC = CH // L


def _embed_body(users, items, u_w, i_w, u_b, i_b, out,
                u_idx, i_idx, u_r0, u_r1, i_r0, i_r1, ub_v, ib_v, res_v,
                sem_u0, sem_u1, sem_i0, sem_i1, sem_b):
    wid = lax.axis_index("s") * NC + lax.axis_index("c")
    base = wid * BPW

    for c in range(NCHUNK):
        pltpu.sync_copy(users.at[pl.ds(base + c * CH, CH)], u_idx.at[c])
        pltpu.sync_copy(items.at[pl.ds(base + c * CH, CH)], i_idx.at[c])

    bias_cps = []
    for c in range(NCHUNK):
        bias_cps.append(
            pltpu.async_copy(u_b.at[u_idx.at[c]], ub_v.at[pl.ds(c * CH, CH)], sem_b))
        bias_cps.append(
            pltpu.async_copy(i_b.at[i_idx.at[c]], ib_v.at[pl.ds(c * CH, CH)], sem_b))

    u_bufs, i_bufs = (u_r0, u_r1), (i_r0, i_r1)
    sem_us, sem_is = (sem_u0, sem_u1), (sem_i0, sem_i1)

    def fire(c):
        bsel = c % 2
        cu = pltpu.async_copy(u_w.at[u_idx.at[c]], u_bufs[bsel], sem_us[bsel])
        ci = pltpu.async_copy(i_w.at[i_idx.at[c]], i_bufs[bsel], sem_is[bsel])
        return cu, ci

    inflight = fire(0)
    for cp in bias_cps:
        cp.wait()

    iota = lax.iota(jnp.int32, L)

    for c in range(NCHUNK):
        cu, ci = inflight
        if c + 1 < NCHUNK:
            nxt = fire(c + 1)
        cu.wait()
        ci.wait()
        if c + 1 < NCHUNK:
            inflight = nxt
        ur, ir = u_bufs[c % 2], i_bufs[c % 2]

        def group(g, carry):
            rows = iota + g * L
            acc = jnp.zeros((L,), jnp.float32)
            for f in range(F):
                col = jnp.full((L,), f, jnp.int32)
                uv = plsc.load_gather(ur, [rows, col])
                iv = plsc.load_gather(ir, [rows, col])
                acc = acc + uv * iv
            off = c * CH + g * L
            x = acc + res_bias_u(off) + res_bias_i(off)
            res_v[pl.ds(off, L)] = 5.0 / (1.0 + jnp.exp(-x))
            return carry

        def res_bias_u(off):
            return ub_v[pl.ds(off, L)]

        def res_bias_i(off):
            return ib_v[pl.ds(off, L)]

        lax.fori_loop(0, GPC, group, 0)

    pltpu.sync_copy(res_v, out.at[pl.ds(base, BPW)])


@jax.jit
def _embed_net(users, items, u_w, i_w, u_b_flat, i_b_flat):
    mesh = plsc.VectorSubcoreMesh(core_axis_name="c", subcore_axis_name="s")
    run = pl.kernel(
        _embed_body,
        out_type=jax.ShapeDtypeStruct((B,), jnp.float32),
        mesh=mesh,
        scratch_types=[
            pltpu.VMEM((NCHUNK, CH), jnp.int32),
            pltpu.VMEM((NCHUNK, CH), jnp.int32),
            pltpu.VMEM((CH, F), jnp.float32),
            pltpu.VMEM((CH, F), jnp.float32),
            pltpu.VMEM((CH, F), jnp.float32),
            pltpu.VMEM((CH, F), jnp.float32),
            pltpu.VMEM((BPW,), jnp.float32),
            pltpu.VMEM((BPW,), jnp.float32),
            pltpu.VMEM((BPW,), jnp.float32),
            pltpu.SemaphoreType.DMA,
            pltpu.SemaphoreType.DMA,
            pltpu.SemaphoreType.DMA,
            pltpu.SemaphoreType.DMA,
            pltpu.SemaphoreType.DMA,
        ],
        compiler_params=pltpu.CompilerParams(needs_layout_passes=False),
    )
    return run(users, items, u_w, i_w, u_b_flat, i_b_flat)


def kernel(users, items, u_weight, i_weight, u_bias, i_bias):
    return _embed_net(users, items, u_weight, i_weight,
                      u_bias.reshape(-1), i_bias.reshape(-1))


# speedup vs baseline: 2.4387x; 2.4387x over previous
import functools

import jax
import jax.numpy as jnp
from jax import lax
from jax.experimental import pallas as pl
from jax.experimental.pallas import tpu as pltpu
from jax.experimental.pallas import tpu_sc as plsc

B = 16384
F = 128
NC, NS, L = 2, 16, 16
NW = NC * NS
BPW = B // NW
CH = 128
NCHUNK = BPW // CH
GPC = CH // L
PADW = L + 1


def _embed_body(users, items, u_w, i_w, u_b, i_b, out,
                u_idx, i_idx, u_r0, u_r1, i_r0, i_r1, ub_v, ib_v, res_v, pad,
                sem_u0, sem_u1, sem_i0, sem_i1, sem_b):
    wid = lax.axis_index("s") * NC + lax.axis_index("c")
    base = wid * BPW

    for c in range(NCHUNK):
        pltpu.sync_copy(users.at[pl.ds(base + c * CH, CH)], u_idx.at[c])
        pltpu.sync_copy(items.at[pl.ds(base + c * CH, CH)], i_idx.at[c])

    bias_cps = []
    for c in range(NCHUNK):
        bias_cps.append(
            pltpu.async_copy(u_b.at[u_idx.at[c]], ub_v.at[pl.ds(c * CH, CH)], sem_b))
        bias_cps.append(
            pltpu.async_copy(i_b.at[i_idx.at[c]], ib_v.at[pl.ds(c * CH, CH)], sem_b))

    u_bufs, i_bufs = (u_r0, u_r1), (i_r0, i_r1)
    sem_us, sem_is = (sem_u0, sem_u1), (sem_i0, sem_i1)

    def fire(c):
        bsel = c % 2
        cu = pltpu.async_copy(u_w.at[u_idx.at[c]], u_bufs[bsel], sem_us[bsel])
        ci = pltpu.async_copy(i_w.at[i_idx.at[c]], i_bufs[bsel], sem_is[bsel])
        return cu, ci

    inflight = {0: fire(0), 1: fire(1)}
    for cp in bias_cps:
        cp.wait()

    iota = lax.iota(jnp.int32, L)
    i_pad = iota * PADW

    for c in range(NCHUNK):
        cu, ci = inflight[c]
        cu.wait()
        ci.wait()
        ur, ir = u_bufs[c % 2], i_bufs[c % 2]

        def group(g, carry):
            for r in range(L):
                row = g * L + r
                acc = ur[row, pl.ds(0, L)] * ir[row, pl.ds(0, L)]
                for j in range(1, F // L):
                    acc = acc + ur[row, pl.ds(j * L, L)] * ir[row, pl.ds(j * L, L)]
                plsc.store_scatter(pad, [i_pad + r], acc)
            tot = plsc.load_gather(pad, [iota])
            for j in range(1, L):
                tot = tot + plsc.load_gather(pad, [iota + j * PADW])
            off = c * CH + g * L
            x = tot + ub_v[pl.ds(off, L)] + ib_v[pl.ds(off, L)]
            res_v[pl.ds(off, L)] = 5.0 / (1.0 + jnp.exp(-x))
            return carry

        lax.fori_loop(0, GPC, group, 0)
        if c + 2 < NCHUNK:
            inflight[c + 2] = fire(c + 2)

    pltpu.sync_copy(res_v, out.at[pl.ds(base, BPW)])


@jax.jit
def _embed_net(users, items, u_w, i_w, u_b_flat, i_b_flat):
    mesh = plsc.VectorSubcoreMesh(core_axis_name="c", subcore_axis_name="s")
    run = pl.kernel(
        _embed_body,
        out_type=jax.ShapeDtypeStruct((B,), jnp.float32),
        mesh=mesh,
        scratch_types=[
            pltpu.VMEM((NCHUNK, CH), jnp.int32),
            pltpu.VMEM((NCHUNK, CH), jnp.int32),
            pltpu.VMEM((CH, F), jnp.float32),
            pltpu.VMEM((CH, F), jnp.float32),
            pltpu.VMEM((CH, F), jnp.float32),
            pltpu.VMEM((CH, F), jnp.float32),
            pltpu.VMEM((BPW,), jnp.float32),
            pltpu.VMEM((BPW,), jnp.float32),
            pltpu.VMEM((BPW,), jnp.float32),
            pltpu.VMEM((L * PADW,), jnp.float32),
            pltpu.SemaphoreType.DMA,
            pltpu.SemaphoreType.DMA,
            pltpu.SemaphoreType.DMA,
            pltpu.SemaphoreType.DMA,
            pltpu.SemaphoreType.DMA,
        ],
        compiler_params=pltpu.CompilerParams(needs_layout_passes=False),
    )
    return run(users, items, u_w, i_w, u_b_flat, i_b_flat)


def kernel(users, items, u_weight, i_weight, u_bias, i_bias):
    return _embed_net(users, items, u_weight, i_weight,
                      u_bias.reshape(-1), i_bias.reshape(-1))


# speedup vs baseline: 2.4866x; 1.0196x over previous
import jax
import jax.numpy as jnp
from jax import lax
from jax.experimental import pallas as pl
from jax.experimental.pallas import tpu as pltpu
from jax.experimental.pallas import tpu_sc as plsc

B = 16384
F = 128
NC, NS, L = 2, 16, 16
NW = NC * NS
BPW = B // NW
CH = 64
NCHUNK = BPW // CH
NBUF = 3
GPC = CH // L
BCH = 128
PADW = L + 1


def _embed_body(users, items, u_w, i_w, u_b, i_b, out,
                u_idx, i_idx, u_bufs, i_bufs, ub_v, ib_v, res_v, pad,
                sem_u, sem_i, sem_b):
    wid = lax.axis_index("s") * NC + lax.axis_index("c")
    base = wid * BPW

    cp_ui = pltpu.async_copy(users.at[pl.ds(base, BPW)], u_idx, sem_b)
    cp_ii = pltpu.async_copy(items.at[pl.ds(base, BPW)], i_idx, sem_b)
    cp_ui.wait()
    cp_ii.wait()

    def fire(c):
        bsel = c % NBUF
        idx_sl = pl.ds(c * CH, CH)
        cu = pltpu.async_copy(u_w.at[u_idx.at[idx_sl]], u_bufs[bsel], sem_u[bsel])
        ci = pltpu.async_copy(i_w.at[i_idx.at[idx_sl]], i_bufs[bsel], sem_i[bsel])
        return cu, ci

    inflight = {c: fire(c) for c in range(NBUF)}

    bias_cps = []
    for c in range(BPW // BCH):
        sl = pl.ds(c * BCH, BCH)
        bias_cps.append(pltpu.async_copy(u_b.at[u_idx.at[sl]], ub_v.at[sl], sem_b))
        bias_cps.append(pltpu.async_copy(i_b.at[i_idx.at[sl]], ib_v.at[sl], sem_b))

    iota = lax.iota(jnp.int32, L)
    i_pad = iota * PADW

    for cp in bias_cps:
        cp.wait()

    for c in range(NCHUNK):
        cu, ci = inflight.pop(c)
        cu.wait()
        ci.wait()
        ur, ir = u_bufs[c % NBUF], i_bufs[c % NBUF]

        def group(g, carry):
            for r in range(L):
                row = g * L + r
                acc = ur[row, pl.ds(0, L)] * ir[row, pl.ds(0, L)]
                for j in range(1, F // L):
                    acc = acc + ur[row, pl.ds(j * L, L)] * ir[row, pl.ds(j * L, L)]
                plsc.store_scatter(pad, [i_pad + r], acc)
            tot = plsc.load_gather(pad, [iota])
            for j in range(1, L):
                tot = tot + plsc.load_gather(pad, [iota + j * PADW])
            off = c * CH + g * L
            x = tot + ub_v[pl.ds(off, L)] + ib_v[pl.ds(off, L)]
            res_v[pl.ds(off, L)] = 5.0 / (1.0 + jnp.exp(-x))
            return carry

        lax.fori_loop(0, GPC, group, 0)
        if c + NBUF < NCHUNK:
            inflight[c + NBUF] = fire(c + NBUF)

    pltpu.sync_copy(res_v, out.at[pl.ds(base, BPW)])


@jax.jit
def _embed_net(users, items, u_w, i_w, u_b_flat, i_b_flat):
    mesh = plsc.VectorSubcoreMesh(core_axis_name="c", subcore_axis_name="s")
    run = pl.kernel(
        _embed_body,
        out_type=jax.ShapeDtypeStruct((B,), jnp.float32),
        mesh=mesh,
        scratch_types=[
            pltpu.VMEM((BPW,), jnp.int32),
            pltpu.VMEM((BPW,), jnp.int32),
            [pltpu.VMEM((CH, F), jnp.float32) for _ in range(NBUF)],
            [pltpu.VMEM((CH, F), jnp.float32) for _ in range(NBUF)],
            pltpu.VMEM((BPW,), jnp.float32),
            pltpu.VMEM((BPW,), jnp.float32),
            pltpu.VMEM((BPW,), jnp.float32),
            pltpu.VMEM((L * PADW,), jnp.float32),
            [pltpu.SemaphoreType.DMA for _ in range(NBUF)],
            [pltpu.SemaphoreType.DMA for _ in range(NBUF)],
            pltpu.SemaphoreType.DMA,
        ],
        compiler_params=pltpu.CompilerParams(needs_layout_passes=False),
    )
    return run(users, items, u_w, i_w, u_b_flat, i_b_flat)


def kernel(users, items, u_weight, i_weight, u_bias, i_bias):
    return _embed_net(users, items, u_weight, i_weight,
                      u_bias.reshape(-1), i_bias.reshape(-1))
